# baseline (speedup 1.0000x reference)
import functools

import jax
import jax.numpy as jnp
from jax import lax
from jax.experimental import pallas as pl
from jax.experimental.pallas import tpu as pltpu

N = 50000
E = 800000
G = 512
D = 128
L = 5
BN_EPS = 1e-5

BR = 512
NPAD = 50176
NBLK = NPAD // BR


def _layer_body(h_ref, agg_ref, w1_ref, b1_ref, w2_ref, b2_ref,
                y0, y1, y2, y3, sum_ref, sumsq_ref):
    pid = pl.program_id(0)
    z = h_ref[...] + agg_ref[...]
    u = jnp.dot(z, w1_ref[...], preferred_element_type=jnp.float32,
                precision=lax.Precision.DEFAULT)
    u = jnp.maximum(u + b1_ref[...], 0.0)
    v = jnp.dot(u, w2_ref[...], preferred_element_type=jnp.float32,
                precision=lax.Precision.DEFAULT)
    y = jnp.maximum(v + b2_ref[...], 0.0)
    y0[...] = y[:, 0:32]
    y1[...] = y[:, 32:64]
    y2[...] = y[:, 64:96]
    y3[...] = y[:, 96:128]
    row = pid * BR + lax.broadcasted_iota(jnp.int32, (BR, 1), 0)
    ym = jnp.where(row < N, y, 0.0)
    ps = jnp.sum(ym.reshape(BR // 8, 8, D), axis=0)
    ps2 = jnp.sum((ym * ym).reshape(BR // 8, 8, D), axis=0)

    @pl.when(pid == 0)
    def _():
        sum_ref[...] = ps
        sumsq_ref[...] = ps2

    @pl.when(pid != 0)
    def _():
        sum_ref[...] += ps
        sumsq_ref[...] += ps2


def _tc_layer(h, agg, w1, b1, w2, b2):
    blk = lambda c: pl.BlockSpec((BR, c), lambda i: (i, 0))
    rep = lambda r, c: pl.BlockSpec((r, c), lambda i: (0, 0))
    out = jax.ShapeDtypeStruct((NPAD, 32), jnp.float32)
    return pl.pallas_call(
        _layer_body,
        grid=(NBLK,),
        in_specs=[blk(D), blk(D), rep(D, D), rep(1, D), rep(D, D), rep(1, D)],
        out_specs=[blk(32), blk(32), blk(32), blk(32), rep(8, D), rep(8, D)],
        out_shape=[out, out, out, out,
                   jax.ShapeDtypeStruct((8, D), jnp.float32),
                   jax.ShapeDtypeStruct((8, D), jnp.float32)],
    )(h, agg, w1, b1, w2, b2)


def _norm_body(y0, y1, y2, y3, sum_ref, sumsq_ref, g_ref, b_ref,
               z0, z1, z2, z3):
    mean = jnp.sum(sum_ref[...], axis=0, keepdims=True) / N
    var = jnp.sum(sumsq_ref[...], axis=0, keepdims=True) / N - mean * mean
    rstd = lax.rsqrt(var + BN_EPS)
    scale = g_ref[...] * rstd
    shift = b_ref[...] - mean * scale
    for i, (yr, zr) in enumerate(((y0, z0), (y1, z1), (y2, z2), (y3, z3))):
        zr[...] = yr[...] * scale[:, 32 * i:32 * i + 32] \
            + shift[:, 32 * i:32 * i + 32]


def _tc_norm(ys, sum8, sumsq8, gamma, beta):
    blk32 = pl.BlockSpec((BR, 32), lambda i: (i, 0))
    rep = lambda r, c: pl.BlockSpec((r, c), lambda i: (0, 0))
    out = jax.ShapeDtypeStruct((NPAD, 32), jnp.float32)
    return pl.pallas_call(
        _norm_body,
        grid=(NBLK,),
        in_specs=[blk32] * 4 + [rep(8, D), rep(8, D), rep(1, D), rep(1, D)],
        out_specs=[blk32] * 4,
        out_shape=[out] * 4,
    )(*ys, sum8, sumsq8, gamma, beta)


def kernel(x, edge_index, batch, params):
    src = edge_index[0]
    dst = edge_index[1]

    x = jnp.pad(x, ((0, NPAD - N), (0, D - x.shape[1])))
    w1_0 = jnp.pad(params[0]["W1"], ((0, D - params[0]["W1"].shape[0]), (0, 0)))

    h = x
    pooled = []
    for i, p in enumerate(params):
        w1 = w1_0 if i == 0 else p["W1"]
        hn = h[:N]
        agg = jnp.zeros((N, D), jnp.float32).at[dst].add(hn[src])
        agg = jnp.pad(agg, ((0, NPAD - N), (0, 0)))
        ys0, ys1, ys2, ys3, s8, ss8 = _tc_layer(
            h, agg, w1, p["b1"].reshape(1, D), p["W2"], p["b2"].reshape(1, D))
        zs = _tc_norm((ys0, ys1, ys2, ys3), s8, ss8,
                      p["gamma"].reshape(1, D), p["beta"].reshape(1, D))
        h = jnp.concatenate(zs, axis=1)
        pooled.append(jax.ops.segment_max(h[:N], batch, num_segments=G))
    return jnp.concatenate(pooled, axis=1)


# speedup vs baseline: 3.6217x; 3.6217x over previous
import functools

import jax
import jax.numpy as jnp
from jax import lax
from jax.experimental import pallas as pl
from jax.experimental.pallas import tpu as pltpu
from jax.experimental.pallas import tpu_sc as plsc

N = 50000
E = 800000
G = 512
D = 128
L = 5
BN_EPS = 1e-5

BR = 512
NPAD = 50176
NBLK = NPAD // BR
TRASH = NPAD - 1

NSC = 2
NT = 16
EPT = 50176
NCH = 196
CJ = 2
RPT = NPAD // NT


def _agg_body(h0, h1, h2, h3, srcR, dstR, a0, a1, a2, a3,
              acc, zbuf, sv0, sv1, dv0, dv1, r0, r1, si0, si1, sg0, sg1):
    c = lax.axis_index("c")
    s = lax.axis_index("s")

    @pl.loop(0, 196)
    def _(i):
        zbuf[i, 0:16] = jnp.zeros((16,), jnp.float32)
        zbuf[i, 16:32] = jnp.zeros((16,), jnp.float32)

    def run_pass(h_g, agg_g):
        for q in range(16):
            pltpu.sync_copy(zbuf, acc.at[pl.ds(s * RPT + q * 196, 196)])
        plsc.subcore_barrier()

        def fire_idx(i, sv, dv, sem):
            pltpu.async_copy(srcR.at[s, i], sv, sem)
            pltpu.async_copy(dstR.at[s, i], dv, sem)

        def wait_idx(i, sv, dv, sem):
            pltpu.make_async_copy(srcR.at[s, i], sv, sem).wait()
            pltpu.make_async_copy(dstR.at[s, i], dv, sem).wait()

        def fire_g(sv, rows, sem):
            for j in range(CJ):
                pltpu.async_copy(h_g.at[sv.at[j]],
                                 rows.at[pl.ds(j * 128, 128)], sem)

        def wait_g(rows, sem):
            pltpu.make_async_copy(h_g.at[pl.ds(0, CJ * 128)], rows, sem).wait()

        def scat(dv, rows):
            for j in range(CJ):
                pltpu.sync_copy(rows.at[pl.ds(j * 128, 128)],
                                acc.at[dv.at[j]], add=True)

        fire_idx(0, sv0, dv0, si0)
        wait_idx(0, sv0, dv0, si0)
        fire_g(sv0, r0, sg0)
        fire_idx(1, sv1, dv1, si1)

        @pl.loop(0, NCH, step=2)
        def _(i):
            wait_idx(i + 1, sv1, dv1, si1)
            fire_g(sv1, r1, sg1)
            wait_g(r0, sg0)
            scat(dv0, r0)

            @pl.when(i + 2 < NCH)
            def _():
                fire_idx(i + 2, sv0, dv0, si0)
                wait_idx(i + 2, sv0, dv0, si0)
                fire_g(sv0, r0, sg0)

            wait_g(r1, sg1)
            scat(dv1, r1)

            @pl.when(i + 3 < NCH)
            def _():
                fire_idx(i + 3, sv1, dv1, si1)

        plsc.subcore_barrier()
        pltpu.sync_copy(acc.at[pl.ds(s * RPT, RPT)],
                        agg_g.at[pl.ds(s * RPT, RPT)])
        plsc.subcore_barrier()

    hs = (h0, h1, h2, h3)
    outs = (a0, a1, a2, a3)
    for core_id in range(2):
        @pl.when(c == core_id)
        def _(core_id=core_id):
            for gi in range(2):
                g = 2 * core_id + gi
                run_pass(hs[g], outs[g])


def _sc_agg(hs, srcR, dstR):
    mesh = plsc.VectorSubcoreMesh(core_axis_name="c", subcore_axis_name="s")
    out = jax.ShapeDtypeStruct((NPAD, 32), jnp.float32)
    fn = pl.kernel(
        _agg_body,
        out_type=[out] * 4,
        mesh=mesh,
        compiler_params=pltpu.CompilerParams(use_tc_tiling_on_sc=False),
        scratch_types=[
            pltpu.VMEM_SHARED((NPAD, 32), jnp.float32),
            pltpu.VMEM((196, 32), jnp.float32),
            pltpu.VMEM((CJ, 128), jnp.int32),
            pltpu.VMEM((CJ, 128), jnp.int32),
            pltpu.VMEM((CJ, 128), jnp.int32),
            pltpu.VMEM((CJ, 128), jnp.int32),
            pltpu.VMEM((CJ * 128, 32), jnp.float32),
            pltpu.VMEM((CJ * 128, 32), jnp.float32),
            pltpu.SemaphoreType.DMA,
            pltpu.SemaphoreType.DMA,
            pltpu.SemaphoreType.DMA,
            pltpu.SemaphoreType.DMA,
        ],
    )
    return fn(*hs, srcR, dstR)


def _layer_body(h0, h1, h2, h3, a0, a1, a2, a3, w1_ref, b1_ref, w2_ref,
                b2_ref, y0, y1, y2, y3, sum_ref, sumsq_ref):
    pid = pl.program_id(0)
    u = jnp.zeros((BR, D), jnp.float32)
    for g, (hr, ar) in enumerate(((h0, a0), (h1, a1), (h2, a2), (h3, a3))):
        z = hr[...] + ar[...]
        u = u + jnp.dot(z, w1_ref[32 * g:32 * g + 32, :],
                        preferred_element_type=jnp.float32)
    u = jnp.maximum(u + b1_ref[...], 0.0)
    v = jnp.dot(u, w2_ref[...], preferred_element_type=jnp.float32)
    y = jnp.maximum(v + b2_ref[...], 0.0)
    y0[...] = y[:, 0:32]
    y1[...] = y[:, 32:64]
    y2[...] = y[:, 64:96]
    y3[...] = y[:, 96:128]
    row = pid * BR + lax.broadcasted_iota(jnp.int32, (BR, 1), 0)
    ym = jnp.where(row < N, y, 0.0)
    ps = jnp.sum(ym.reshape(BR // 8, 8, D), axis=0)
    ps2 = jnp.sum((ym * ym).reshape(BR // 8, 8, D), axis=0)

    @pl.when(pid == 0)
    def _():
        sum_ref[...] = ps
        sumsq_ref[...] = ps2

    @pl.when(pid != 0)
    def _():
        sum_ref[...] += ps
        sumsq_ref[...] += ps2


def _tc_layer(hs, aggs, w1, b1, w2, b2):
    blk32 = pl.BlockSpec((BR, 32), lambda i: (i, 0))
    rep = lambda r, c: pl.BlockSpec((r, c), lambda i: (0, 0))
    out = jax.ShapeDtypeStruct((NPAD, 32), jnp.float32)
    return pl.pallas_call(
        _layer_body,
        grid=(NBLK,),
        in_specs=[blk32] * 8 + [rep(D, D), rep(1, D), rep(D, D), rep(1, D)],
        out_specs=[blk32] * 4 + [rep(8, D), rep(8, D)],
        out_shape=[out, out, out, out,
                   jax.ShapeDtypeStruct((8, D), jnp.float32),
                   jax.ShapeDtypeStruct((8, D), jnp.float32)],
    )(*hs, *aggs, w1, b1, w2, b2)


def _norm_body(y0, y1, y2, y3, sum_ref, sumsq_ref, g_ref, b_ref,
               z0, z1, z2, z3):
    mean = jnp.sum(sum_ref[...], axis=0, keepdims=True) / N
    var = jnp.sum(sumsq_ref[...], axis=0, keepdims=True) / N - mean * mean
    rstd = lax.rsqrt(var + BN_EPS)
    scale = g_ref[...] * rstd
    shift = b_ref[...] - mean * scale
    for i, (yr, zr) in enumerate(((y0, z0), (y1, z1), (y2, z2), (y3, z3))):
        zr[...] = yr[...] * scale[:, 32 * i:32 * i + 32] \
            + shift[:, 32 * i:32 * i + 32]


def _tc_norm(ys, sum8, sumsq8, gamma, beta):
    blk32 = pl.BlockSpec((BR, 32), lambda i: (i, 0))
    rep = lambda r, c: pl.BlockSpec((r, c), lambda i: (0, 0))
    out = jax.ShapeDtypeStruct((NPAD, 32), jnp.float32)
    return pl.pallas_call(
        _norm_body,
        grid=(NBLK,),
        in_specs=[blk32] * 4 + [rep(8, D), rep(8, D), rep(1, D), rep(1, D)],
        out_specs=[blk32] * 4,
        out_shape=[out] * 4,
    )(*ys, sum8, sumsq8, gamma, beta)


def kernel(x, edge_index, batch, params):
    src = edge_index[0]
    dst = edge_index[1]

    tot = NT * EPT
    pad_n = tot - E
    src_p = jnp.concatenate(
        [src, (jnp.arange(pad_n, dtype=jnp.int32) % N)])
    dst_p = jnp.concatenate(
        [dst, jnp.full((pad_n,), TRASH, dtype=jnp.int32)])
    srcR = src_p.reshape(NT, NCH, CJ, 128)
    dstR = dst_p.reshape(NT, NCH, CJ, 128)

    x = jnp.pad(x, ((0, NPAD - N), (0, D - x.shape[1])))
    w1_0 = jnp.pad(params[0]["W1"], ((0, D - params[0]["W1"].shape[0]), (0, 0)))

    hs = tuple(x[:, 32 * i:32 * i + 32] for i in range(4))
    pooled = []
    for i, p in enumerate(params):
        w1 = w1_0 if i == 0 else p["W1"]
        aggs = _sc_agg(hs, srcR, dstR)
        ys0, ys1, ys2, ys3, s8, ss8 = _tc_layer(
            hs, aggs, w1, p["b1"].reshape(1, D), p["W2"],
            p["b2"].reshape(1, D))
        zs = _tc_norm((ys0, ys1, ys2, ys3), s8, ss8,
                      p["gamma"].reshape(1, D), p["beta"].reshape(1, D))
        hs = tuple(zs)
        h_full = jnp.concatenate(zs, axis=1)
        pooled.append(jax.ops.segment_max(h_full[:N], batch, num_segments=G))
    return jnp.concatenate(pooled, axis=1)


# speedup vs baseline: 4.2620x; 1.1768x over previous
import functools

import jax
import jax.numpy as jnp
from jax import lax
from jax.experimental import pallas as pl
from jax.experimental.pallas import tpu as pltpu
from jax.experimental.pallas import tpu_sc as plsc

N = 50000
E = 800000
G = 512
D = 128
L = 5
BN_EPS = 1e-5

BR = 512
NPAD = 50176
NBLK = NPAD // BR
TRASH = NPAD - 1

NSC = 2
NT = 16
EPT = 50176
NCH = 196
CJ = 2
RPT = NPAD // NT


def _agg_body(h0, h1, h2, h3, srcR, dstR, a0, a1, a2, a3,
              acc, zbuf, sv0, sv1, dv0, dv1, r0, r1, si0, si1, sg0, sg1):
    c = lax.axis_index("c")
    s = lax.axis_index("s")

    @pl.loop(0, 196)
    def _(i):
        zbuf[i, 0:16] = jnp.zeros((16,), jnp.float32)
        zbuf[i, 16:32] = jnp.zeros((16,), jnp.float32)

    def run_pass(h_g, agg_g):
        for q in range(16):
            pltpu.sync_copy(zbuf, acc.at[pl.ds(s * RPT + q * 196, 196)])
        plsc.subcore_barrier()

        def fire_idx(i, sv, dv, sem):
            pltpu.async_copy(srcR.at[s, i], sv, sem)
            pltpu.async_copy(dstR.at[s, i], dv, sem)

        def wait_idx(i, sv, dv, sem):
            pltpu.make_async_copy(srcR.at[s, i], sv, sem).wait()
            pltpu.make_async_copy(dstR.at[s, i], dv, sem).wait()

        def fire_g(sv, rows, sem):
            for j in range(CJ):
                pltpu.async_copy(h_g.at[sv.at[j]],
                                 rows.at[pl.ds(j * 128, 128)], sem)

        def wait_g(rows, sem):
            pltpu.make_async_copy(h_g.at[pl.ds(0, CJ * 128)], rows, sem).wait()

        def scat(dv, rows):
            for j in range(CJ):
                pltpu.sync_copy(rows.at[pl.ds(j * 128, 128)],
                                acc.at[dv.at[j]], add=True)

        fire_idx(0, sv0, dv0, si0)
        wait_idx(0, sv0, dv0, si0)
        fire_g(sv0, r0, sg0)
        fire_idx(1, sv1, dv1, si1)

        @pl.loop(0, NCH, step=2)
        def _(i):
            wait_idx(i + 1, sv1, dv1, si1)
            fire_g(sv1, r1, sg1)
            wait_g(r0, sg0)
            scat(dv0, r0)

            @pl.when(i + 2 < NCH)
            def _():
                fire_idx(i + 2, sv0, dv0, si0)
                wait_idx(i + 2, sv0, dv0, si0)
                fire_g(sv0, r0, sg0)

            wait_g(r1, sg1)
            scat(dv1, r1)

            @pl.when(i + 3 < NCH)
            def _():
                fire_idx(i + 3, sv1, dv1, si1)

        plsc.subcore_barrier()
        pltpu.sync_copy(acc.at[pl.ds(s * RPT, RPT)],
                        agg_g.at[pl.ds(s * RPT, RPT)])
        plsc.subcore_barrier()

    hs = (h0, h1, h2, h3)
    outs = (a0, a1, a2, a3)
    for core_id in range(2):
        @pl.when(c == core_id)
        def _(core_id=core_id):
            for gi in range(2):
                g = 2 * core_id + gi
                run_pass(hs[g], outs[g])


def _sc_agg(hs, srcR, dstR):
    mesh = plsc.VectorSubcoreMesh(core_axis_name="c", subcore_axis_name="s")
    out = jax.ShapeDtypeStruct((NPAD, 32), jnp.float32)
    fn = pl.kernel(
        _agg_body,
        out_type=[out] * 4,
        mesh=mesh,
        compiler_params=pltpu.CompilerParams(use_tc_tiling_on_sc=False),
        scratch_types=[
            pltpu.VMEM_SHARED((NPAD, 32), jnp.float32),
            pltpu.VMEM((196, 32), jnp.float32),
            pltpu.VMEM((CJ, 128), jnp.int32),
            pltpu.VMEM((CJ, 128), jnp.int32),
            pltpu.VMEM((CJ, 128), jnp.int32),
            pltpu.VMEM((CJ, 128), jnp.int32),
            pltpu.VMEM((CJ * 128, 32), jnp.float32),
            pltpu.VMEM((CJ * 128, 32), jnp.float32),
            pltpu.SemaphoreType.DMA,
            pltpu.SemaphoreType.DMA,
            pltpu.SemaphoreType.DMA,
            pltpu.SemaphoreType.DMA,
        ],
    )
    return fn(*hs, srcR, dstR)


BIG = 2 ** 30
CH = 256
NV = NPAD // 16


def _pool_body(*args):
    zs = args[:20]
    batchR = args[20]
    out = args[21]
    (bb, st, rb0, rb1, rb2, rb3, outl, sem) = args[22:]
    rbufs = (rb0, rb1, rb2, rb3)
    c = lax.axis_index("c")
    s = lax.axis_index("s")
    w = c * 16 + s
    iota = lax.iota(jnp.int32, 16)

    pltpu.sync_copy(batchR, bb)

    @pl.loop(0, 33)
    def _(k):
        st[pl.ds(16 * k, 16)] = jnp.full((16,), BIG, jnp.int32)

    def bscan(i, carry):
        v = bb[pl.ds(16 * i, 16)]
        prev = jnp.take(v, jnp.maximum(iota - 1, 0))
        prev = jnp.where(iota == 0, carry, prev)
        m = v != prev
        rowidx = jnp.full((16,), 16 * i, jnp.int32) + iota
        plsc.store_scatter(st, [v], rowidx, mask=m)
        return jnp.max(jnp.where(iota == 15, v, -1))

    lax.fori_loop(0, NV, bscan, jnp.int32(-1))

    def bfill(k, carry):
        kk = 32 - k
        v = st[pl.ds(16 * kk, 16)]
        cm = plsc.cummax(lax.rev(-v, (0,)))
        cm = jnp.maximum(cm, jnp.broadcast_to(carry, (16,)))
        st[pl.ds(16 * kk, 16)] = lax.rev(-cm, (0,))
        return jnp.max(cm)

    lax.fori_loop(0, 33, bfill, jnp.int32(-N))

    def extract(vec, k):
        return jnp.max(jnp.where(iota == k, vec, -BIG))

    v_st = st[pl.ds(16 * w, 16)]
    v_en = st[pl.ds(16 * w + 8, 16)]

    @pl.loop(0, 16)
    def _(gl):
        g0 = extract(v_st, gl)
        g1 = jnp.where(gl < 15, extract(v_st, gl + 1), extract(v_en, 8))
        nrows = g1 - g0
        nch = (nrows + (CH - 1)) // CH
        for l in range(5):
            def chunk(ci, acc):
                base = g0 + ci * CH
                for q in range(4):
                    pltpu.async_copy(zs[4 * l + q].at[pl.ds(base, CH)],
                                     rbufs[q], sem)
                for q in range(4):
                    pltpu.make_async_copy(zs[4 * l + q].at[pl.ds(base, CH)],
                                          rbufs[q], sem).wait()
                cnt = jnp.minimum(nrows - ci * CH, CH)

                def row(rr, a):
                    return tuple(
                        jnp.maximum(a[2 * q + h],
                                    rbufs[q][rr, 16 * h:16 * h + 16])
                        for q in range(4) for h in range(2))

                return lax.fori_loop(0, cnt, row, acc)

            ninf = jnp.full((16,), -jnp.inf, jnp.float32)
            acc = lax.fori_loop(0, nch, chunk, (ninf,) * 8)
            for q in range(4):
                for h in range(2):
                    outl[gl, pl.ds(128 * l + 32 * q + 16 * h, 16)] = \
                        acc[2 * q + h]

    pltpu.sync_copy(outl, out.at[pl.ds(16 * w, 16)])


def _sc_pool(z_all, batchR):
    mesh = plsc.VectorSubcoreMesh(core_axis_name="c", subcore_axis_name="s")
    fn = pl.kernel(
        _pool_body,
        out_type=jax.ShapeDtypeStruct((G, 5 * D), jnp.float32),
        mesh=mesh,
        compiler_params=pltpu.CompilerParams(use_tc_tiling_on_sc=False,
                                             needs_layout_passes=False),
        scratch_types=[
            pltpu.VMEM((NPAD,), jnp.int32),
            pltpu.VMEM((528,), jnp.int32),
            pltpu.VMEM((CH, 32), jnp.float32),
            pltpu.VMEM((CH, 32), jnp.float32),
            pltpu.VMEM((CH, 32), jnp.float32),
            pltpu.VMEM((CH, 32), jnp.float32),
            pltpu.VMEM((16, 640), jnp.float32),
            pltpu.SemaphoreType.DMA,
        ],
    )
    return fn(*z_all, batchR)


def _layer_body(h0, h1, h2, h3, a0, a1, a2, a3, w1_ref, b1_ref, w2_ref,
                b2_ref, y0, y1, y2, y3, sum_ref, sumsq_ref):
    pid = pl.program_id(0)
    u = jnp.zeros((BR, D), jnp.float32)
    for g, (hr, ar) in enumerate(((h0, a0), (h1, a1), (h2, a2), (h3, a3))):
        z = hr[...] + ar[...]
        u = u + jnp.dot(z, w1_ref[32 * g:32 * g + 32, :],
                        preferred_element_type=jnp.float32)
    u = jnp.maximum(u + b1_ref[...], 0.0)
    v = jnp.dot(u, w2_ref[...], preferred_element_type=jnp.float32)
    y = jnp.maximum(v + b2_ref[...], 0.0)
    y0[...] = y[:, 0:32]
    y1[...] = y[:, 32:64]
    y2[...] = y[:, 64:96]
    y3[...] = y[:, 96:128]
    row = pid * BR + lax.broadcasted_iota(jnp.int32, (BR, 1), 0)
    ym = jnp.where(row < N, y, 0.0)
    ps = jnp.sum(ym.reshape(BR // 8, 8, D), axis=0)
    ps2 = jnp.sum((ym * ym).reshape(BR // 8, 8, D), axis=0)

    @pl.when(pid == 0)
    def _():
        sum_ref[...] = ps
        sumsq_ref[...] = ps2

    @pl.when(pid != 0)
    def _():
        sum_ref[...] += ps
        sumsq_ref[...] += ps2


def _tc_layer(hs, aggs, w1, b1, w2, b2):
    blk32 = pl.BlockSpec((BR, 32), lambda i: (i, 0))
    rep = lambda r, c: pl.BlockSpec((r, c), lambda i: (0, 0))
    out = jax.ShapeDtypeStruct((NPAD, 32), jnp.float32)
    return pl.pallas_call(
        _layer_body,
        grid=(NBLK,),
        in_specs=[blk32] * 8 + [rep(D, D), rep(1, D), rep(D, D), rep(1, D)],
        out_specs=[blk32] * 4 + [rep(8, D), rep(8, D)],
        out_shape=[out, out, out, out,
                   jax.ShapeDtypeStruct((8, D), jnp.float32),
                   jax.ShapeDtypeStruct((8, D), jnp.float32)],
    )(*hs, *aggs, w1, b1, w2, b2)


def _norm_body(y0, y1, y2, y3, sum_ref, sumsq_ref, g_ref, b_ref,
               z0, z1, z2, z3):
    mean = jnp.sum(sum_ref[...], axis=0, keepdims=True) / N
    var = jnp.sum(sumsq_ref[...], axis=0, keepdims=True) / N - mean * mean
    rstd = lax.rsqrt(var + BN_EPS)
    scale = g_ref[...] * rstd
    shift = b_ref[...] - mean * scale
    for i, (yr, zr) in enumerate(((y0, z0), (y1, z1), (y2, z2), (y3, z3))):
        zr[...] = yr[...] * scale[:, 32 * i:32 * i + 32] \
            + shift[:, 32 * i:32 * i + 32]


def _tc_norm(ys, sum8, sumsq8, gamma, beta):
    blk32 = pl.BlockSpec((BR, 32), lambda i: (i, 0))
    rep = lambda r, c: pl.BlockSpec((r, c), lambda i: (0, 0))
    out = jax.ShapeDtypeStruct((NPAD, 32), jnp.float32)
    return pl.pallas_call(
        _norm_body,
        grid=(NBLK,),
        in_specs=[blk32] * 4 + [rep(8, D), rep(8, D), rep(1, D), rep(1, D)],
        out_specs=[blk32] * 4,
        out_shape=[out] * 4,
    )(*ys, sum8, sumsq8, gamma, beta)


def kernel(x, edge_index, batch, params):
    src = edge_index[0]
    dst = edge_index[1]

    tot = NT * EPT
    pad_n = tot - E
    src_p = jnp.concatenate(
        [src, (jnp.arange(pad_n, dtype=jnp.int32) % N)])
    dst_p = jnp.concatenate(
        [dst, jnp.full((pad_n,), TRASH, dtype=jnp.int32)])
    srcR = src_p.reshape(NT, NCH, CJ, 128)
    dstR = dst_p.reshape(NT, NCH, CJ, 128)

    x = jnp.pad(x, ((0, NPAD - N), (0, D - x.shape[1])))
    w1_0 = jnp.pad(params[0]["W1"], ((0, D - params[0]["W1"].shape[0]), (0, 0)))

    batchR = jnp.pad(batch, (0, NPAD - N), constant_values=G)

    hs = tuple(x[:, 32 * i:32 * i + 32] for i in range(4))
    z_all = []
    for i, p in enumerate(params):
        w1 = w1_0 if i == 0 else p["W1"]
        aggs = _sc_agg(hs, srcR, dstR)
        ys0, ys1, ys2, ys3, s8, ss8 = _tc_layer(
            hs, aggs, w1, p["b1"].reshape(1, D), p["W2"],
            p["b2"].reshape(1, D))
        zs = _tc_norm((ys0, ys1, ys2, ys3), s8, ss8,
                      p["gamma"].reshape(1, D), p["beta"].reshape(1, D))
        hs = tuple(zs)
        z_all.extend(zs)
    return _sc_pool(z_all, batchR)


# speedup vs baseline: 4.8872x; 1.1467x over previous
import functools

import jax
import jax.numpy as jnp
from jax import lax
from jax.experimental import pallas as pl
from jax.experimental.pallas import tpu as pltpu
from jax.experimental.pallas import tpu_sc as plsc

N = 50000
E = 800000
G = 512
D = 128
L = 5
BN_EPS = 1e-5

BR = 512
NPAD = 50176
NBLK = NPAD // BR
TRASH = NPAD - 1

NSC = 2
NT = 16
EPT = 50176
NCH = 196
CJ = 2
RPT = NPAD // NT


def _agg_body(h0, h1, h2, h3, idxR, a0, a1, a2, a3,
              acc, zbuf, iv0, iv1, iv2, iv3, r0, r1,
              si0, si1, si2, si3, sg0, sg1, ss0, ss1, ss2, ss3):
    c = lax.axis_index("c")
    s = lax.axis_index("s")
    ivs = (iv0, iv1, iv2, iv3)
    sis = (si0, si1, si2, si3)
    sss = (ss0, ss1, ss2, ss3)
    rows = (r0, r1)
    sgs = (sg0, sg1)

    @pl.loop(0, 196)
    def _(i):
        zbuf[i, 0:16] = jnp.zeros((16,), jnp.float32)
        zbuf[i, 16:32] = jnp.zeros((16,), jnp.float32)

    def run_pass(h_g, agg_g):
        for q in range(16):
            pltpu.sync_copy(zbuf, acc.at[pl.ds(s * RPT + q * 196, 196)])
        plsc.subcore_barrier()

        def fire_idx(k, b):
            pltpu.async_copy(idxR.at[s, k], ivs[b], sis[b])

        def wait_idx(k, b):
            pltpu.make_async_copy(idxR.at[s, k], ivs[b], sis[b]).wait()

        def fire_g(k, b4, b2):
            for j in range(CJ):
                pltpu.async_copy(h_g.at[ivs[b4].at[2 * j]],
                                 rows[b2].at[pl.ds(j * 128, 128)], sgs[b2])

        def wait_g(b2):
            pltpu.make_async_copy(h_g.at[pl.ds(0, CJ * 128)], rows[b2],
                                  sgs[b2]).wait()

        def fire_scat(b4, b2):
            for j in range(CJ):
                pltpu.async_copy(rows[b2].at[pl.ds(j * 128, 128)],
                                 acc.at[ivs[b4].at[2 * j + 1]], sss[b4],
                                 add=True)

        def wait_scat(b4, b2):
            for j in range(CJ):
                pltpu.make_async_copy(rows[b2].at[pl.ds(j * 128, 128)],
                                      acc.at[ivs[b4].at[2 * j + 1]],
                                      sss[b4]).wait()

        fire_idx(0, 0)
        fire_idx(1, 1)
        wait_idx(0, 0)
        fire_g(0, 0, 0)

        @pl.loop(0, NCH, step=4)
        def _(i):
            for p in range(4):
                k = i + p

                @pl.when(k >= 1)
                def _(p=p):
                    wait_scat((p + 3) % 4, (p + 1) % 2)

                @pl.when(k + 1 < NCH)
                def _(p=p, k=k):
                    wait_idx(k + 1, (p + 1) % 4)
                    fire_g(k + 1, (p + 1) % 4, (p + 1) % 2)

                @pl.when(k + 2 < NCH)
                def _(p=p, k=k):
                    fire_idx(k + 2, (p + 2) % 4)

                wait_g(p % 2)
                fire_scat(p, p % 2)

        wait_scat(3, 1)
        plsc.subcore_barrier()
        pltpu.sync_copy(acc.at[pl.ds(s * RPT, RPT)],
                        agg_g.at[pl.ds(s * RPT, RPT)])
        plsc.subcore_barrier()

    hs = (h0, h1, h2, h3)
    outs = (a0, a1, a2, a3)
    for core_id in range(2):
        @pl.when(c == core_id)
        def _(core_id=core_id):
            for gi in range(2):
                g = 2 * core_id + gi
                run_pass(hs[g], outs[g])


def _sc_agg(hs, idxR):
    mesh = plsc.VectorSubcoreMesh(core_axis_name="c", subcore_axis_name="s")
    out = jax.ShapeDtypeStruct((NPAD, 32), jnp.float32)
    fn = pl.kernel(
        _agg_body,
        out_type=[out] * 4,
        mesh=mesh,
        compiler_params=pltpu.CompilerParams(use_tc_tiling_on_sc=False),
        scratch_types=[
            pltpu.VMEM_SHARED((NPAD, 32), jnp.float32),
            pltpu.VMEM((196, 32), jnp.float32),
            pltpu.VMEM((2 * CJ, 128), jnp.int32),
            pltpu.VMEM((2 * CJ, 128), jnp.int32),
            pltpu.VMEM((2 * CJ, 128), jnp.int32),
            pltpu.VMEM((2 * CJ, 128), jnp.int32),
            pltpu.VMEM((CJ * 128, 32), jnp.float32),
            pltpu.VMEM((CJ * 128, 32), jnp.float32),
            pltpu.SemaphoreType.DMA,
            pltpu.SemaphoreType.DMA,
            pltpu.SemaphoreType.DMA,
            pltpu.SemaphoreType.DMA,
            pltpu.SemaphoreType.DMA,
            pltpu.SemaphoreType.DMA,
            pltpu.SemaphoreType.DMA,
            pltpu.SemaphoreType.DMA,
            pltpu.SemaphoreType.DMA,
            pltpu.SemaphoreType.DMA,
        ],
    )
    return fn(*hs, idxR)


BIG = 2 ** 30
CH = 256
NV = NPAD // 16


def _pool_body(*args):
    zs = args[:20]
    batchR = args[20]
    out = args[21]
    (bb, st, rb0, rb1, rb2, rb3, outl, sem) = args[22:]
    rbufs = (rb0, rb1, rb2, rb3)
    c = lax.axis_index("c")
    s = lax.axis_index("s")
    w = c * 16 + s
    iota = lax.iota(jnp.int32, 16)

    pltpu.sync_copy(batchR, bb)

    @pl.loop(0, 33)
    def _(k):
        st[pl.ds(16 * k, 16)] = jnp.full((16,), BIG, jnp.int32)

    def bscan(i, carry):
        v = bb[pl.ds(16 * i, 16)]
        prev = jnp.take(v, jnp.maximum(iota - 1, 0))
        prev = jnp.where(iota == 0, carry, prev)
        m = v != prev
        rowidx = jnp.full((16,), 16 * i, jnp.int32) + iota
        plsc.store_scatter(st, [v], rowidx, mask=m)
        return jnp.max(jnp.where(iota == 15, v, -1))

    lax.fori_loop(0, NV, bscan, jnp.int32(-1))

    def bfill(k, carry):
        kk = 32 - k
        v = st[pl.ds(16 * kk, 16)]
        cm = plsc.cummax(lax.rev(-v, (0,)))
        cm = jnp.maximum(cm, jnp.broadcast_to(carry, (16,)))
        st[pl.ds(16 * kk, 16)] = lax.rev(-cm, (0,))
        return jnp.max(cm)

    lax.fori_loop(0, 33, bfill, jnp.int32(-N))

    def extract(vec, k):
        return jnp.max(jnp.where(iota == k, vec, -BIG))

    v_st = st[pl.ds(16 * w, 16)]
    v_en = st[pl.ds(16 * w + 8, 16)]

    @pl.loop(0, 16)
    def _(gl):
        g0 = extract(v_st, gl)
        g1 = jnp.where(gl < 15, extract(v_st, gl + 1), extract(v_en, 8))
        nrows = g1 - g0
        nch = (nrows + (CH - 1)) // CH
        for l in range(5):
            def chunk(ci, acc):
                base = g0 + ci * CH
                for q in range(4):
                    pltpu.async_copy(zs[4 * l + q].at[pl.ds(base, CH)],
                                     rbufs[q], sem)
                for q in range(4):
                    pltpu.make_async_copy(zs[4 * l + q].at[pl.ds(base, CH)],
                                          rbufs[q], sem).wait()
                cnt = jnp.minimum(nrows - ci * CH, CH)

                def row(rr, a):
                    return tuple(
                        jnp.maximum(a[2 * q + h],
                                    rbufs[q][rr, 16 * h:16 * h + 16])
                        for q in range(4) for h in range(2))

                return lax.fori_loop(0, cnt, row, acc)

            ninf = jnp.full((16,), -jnp.inf, jnp.float32)
            acc = lax.fori_loop(0, nch, chunk, (ninf,) * 8)
            for q in range(4):
                for h in range(2):
                    outl[gl, pl.ds(128 * l + 32 * q + 16 * h, 16)] = \
                        acc[2 * q + h]

    pltpu.sync_copy(outl, out.at[pl.ds(16 * w, 16)])


def _sc_pool(z_all, batchR):
    mesh = plsc.VectorSubcoreMesh(core_axis_name="c", subcore_axis_name="s")
    fn = pl.kernel(
        _pool_body,
        out_type=jax.ShapeDtypeStruct((G, 5 * D), jnp.float32),
        mesh=mesh,
        compiler_params=pltpu.CompilerParams(use_tc_tiling_on_sc=False,
                                             needs_layout_passes=False),
        scratch_types=[
            pltpu.VMEM((NPAD,), jnp.int32),
            pltpu.VMEM((528,), jnp.int32),
            pltpu.VMEM((CH, 32), jnp.float32),
            pltpu.VMEM((CH, 32), jnp.float32),
            pltpu.VMEM((CH, 32), jnp.float32),
            pltpu.VMEM((CH, 32), jnp.float32),
            pltpu.VMEM((16, 640), jnp.float32),
            pltpu.SemaphoreType.DMA,
        ],
    )
    return fn(*z_all, batchR)


def _layer_body(h0, h1, h2, h3, a0, a1, a2, a3, w1_ref, b1_ref, w2_ref,
                b2_ref, y0, y1, y2, y3, sum_ref, sumsq_ref):
    pid = pl.program_id(0)
    u = jnp.zeros((BR, D), jnp.float32)
    for g, (hr, ar) in enumerate(((h0, a0), (h1, a1), (h2, a2), (h3, a3))):
        z = hr[...] + ar[...]
        u = u + jnp.dot(z, w1_ref[32 * g:32 * g + 32, :],
                        preferred_element_type=jnp.float32)
    u = jnp.maximum(u + b1_ref[...], 0.0)
    v = jnp.dot(u, w2_ref[...], preferred_element_type=jnp.float32)
    y = jnp.maximum(v + b2_ref[...], 0.0)
    y0[...] = y[:, 0:32]
    y1[...] = y[:, 32:64]
    y2[...] = y[:, 64:96]
    y3[...] = y[:, 96:128]
    row = pid * BR + lax.broadcasted_iota(jnp.int32, (BR, 1), 0)
    ym = jnp.where(row < N, y, 0.0)
    ps = jnp.sum(ym.reshape(BR // 8, 8, D), axis=0)
    ps2 = jnp.sum((ym * ym).reshape(BR // 8, 8, D), axis=0)

    @pl.when(pid == 0)
    def _():
        sum_ref[...] = ps
        sumsq_ref[...] = ps2

    @pl.when(pid != 0)
    def _():
        sum_ref[...] += ps
        sumsq_ref[...] += ps2


def _tc_layer(hs, aggs, w1, b1, w2, b2):
    blk32 = pl.BlockSpec((BR, 32), lambda i: (i, 0))
    rep = lambda r, c: pl.BlockSpec((r, c), lambda i: (0, 0))
    out = jax.ShapeDtypeStruct((NPAD, 32), jnp.float32)
    return pl.pallas_call(
        _layer_body,
        grid=(NBLK,),
        in_specs=[blk32] * 8 + [rep(D, D), rep(1, D), rep(D, D), rep(1, D)],
        out_specs=[blk32] * 4 + [rep(8, D), rep(8, D)],
        out_shape=[out, out, out, out,
                   jax.ShapeDtypeStruct((8, D), jnp.float32),
                   jax.ShapeDtypeStruct((8, D), jnp.float32)],
    )(*hs, *aggs, w1, b1, w2, b2)


def _norm_body(y0, y1, y2, y3, sum_ref, sumsq_ref, g_ref, b_ref,
               z0, z1, z2, z3):
    mean = jnp.sum(sum_ref[...], axis=0, keepdims=True) / N
    var = jnp.sum(sumsq_ref[...], axis=0, keepdims=True) / N - mean * mean
    rstd = lax.rsqrt(var + BN_EPS)
    scale = g_ref[...] * rstd
    shift = b_ref[...] - mean * scale
    for i, (yr, zr) in enumerate(((y0, z0), (y1, z1), (y2, z2), (y3, z3))):
        zr[...] = yr[...] * scale[:, 32 * i:32 * i + 32] \
            + shift[:, 32 * i:32 * i + 32]


def _tc_norm(ys, sum8, sumsq8, gamma, beta):
    blk32 = pl.BlockSpec((BR, 32), lambda i: (i, 0))
    rep = lambda r, c: pl.BlockSpec((r, c), lambda i: (0, 0))
    out = jax.ShapeDtypeStruct((NPAD, 32), jnp.float32)
    return pl.pallas_call(
        _norm_body,
        grid=(NBLK,),
        in_specs=[blk32] * 4 + [rep(8, D), rep(8, D), rep(1, D), rep(1, D)],
        out_specs=[blk32] * 4,
        out_shape=[out] * 4,
    )(*ys, sum8, sumsq8, gamma, beta)


def kernel(x, edge_index, batch, params):
    src = edge_index[0]
    dst = edge_index[1]

    tot = NT * EPT
    pad_n = tot - E
    src_p = jnp.concatenate(
        [src, (jnp.arange(pad_n, dtype=jnp.int32) % N)])
    dst_p = jnp.concatenate(
        [dst, jnp.full((pad_n,), TRASH, dtype=jnp.int32)])
    idxR = jnp.stack([src_p.reshape(NT, NCH, CJ, 128),
                      dst_p.reshape(NT, NCH, CJ, 128)], axis=3)
    idxR = idxR.reshape(NT, NCH, 2 * CJ, 128)

    x = jnp.pad(x, ((0, NPAD - N), (0, D - x.shape[1])))
    w1_0 = jnp.pad(params[0]["W1"], ((0, D - params[0]["W1"].shape[0]), (0, 0)))

    batchR = jnp.pad(batch, (0, NPAD - N), constant_values=G)

    hs = tuple(x[:, 32 * i:32 * i + 32] for i in range(4))
    z_all = []
    for i, p in enumerate(params):
        w1 = w1_0 if i == 0 else p["W1"]
        aggs = _sc_agg(hs, idxR)
        ys0, ys1, ys2, ys3, s8, ss8 = _tc_layer(
            hs, aggs, w1, p["b1"].reshape(1, D), p["W2"],
            p["b2"].reshape(1, D))
        zs = _tc_norm((ys0, ys1, ys2, ys3), s8, ss8,
                      p["gamma"].reshape(1, D), p["beta"].reshape(1, D))
        hs = tuple(zs)
        z_all.extend(zs)
    return _sc_pool(z_all, batchR)
